# Initial kernel scaffold; baseline (speedup 1.0000x reference)
import functools

import jax
import jax.numpy as jnp
from jax import lax
from jax.experimental import pallas as pl
from jax.experimental.pallas import tpu as pltpu
from jax.experimental.pallas import tpu_sc as plsc

_NC, _NS, _L = 2, 16, 16
_NW = _NC * _NS
_D = 128
_CTX = 20
_NNEG = 100
_K = 128


def _sc_scores(iword, idx_all, in_W, out_W):
  B = iword.shape[0]
  bpw = B // _NW
  half = bpw // 2
  nd = _D // _L
  ng = _K // _L

  mesh = plsc.VectorSubcoreMesh(core_axis_name="c", subcore_axis_name="s")

  @functools.partial(
      pl.kernel,
      out_type=jax.ShapeDtypeStruct((B, _K), jnp.float32),
      mesh=mesh,
      scratch_types=[
          pltpu.VMEM((bpw,), jnp.int32),
          pltpu.VMEM((bpw, _K), jnp.int32),
          pltpu.VMEM((bpw, _D), jnp.float32),
          pltpu.VMEM((2, _K, _D), jnp.float32),
          pltpu.VMEM((_L, _L), jnp.float32),
          pltpu.VMEM((bpw, _K), jnp.float32),
          pltpu.SemaphoreType.DMA,
          pltpu.SemaphoreType.DMA,
          pltpu.SemaphoreType.DMA,
      ],
  )
  def sc_kernel(iword_hbm, idx_hbm, inw_hbm, outw_hbm, out_hbm,
                iw_v, idx_v, ivec_v, rows_v, tr_v, sc_v, sem0, sem1, semi):
    wid = lax.axis_index("s") * _NC + lax.axis_index("c")
    base = wid * bpw
    pltpu.sync_copy(iword_hbm.at[pl.ds(base, bpw)], iw_v)
    pltpu.sync_copy(idx_hbm.at[pl.ds(base, bpw), :], idx_v)
    pltpu.async_copy(inw_hbm.at[iw_v], ivec_v, semi).wait()

    lane = lax.iota(jnp.int32, _L)

    def compute(b, buf):
      iv = [ivec_v[b, pl.ds(k * _L, _L)] for k in range(nd)]
      for r in range(ng):
        for j in range(_L):
          row = r * _L + j
          acc = rows_v[buf, row, pl.ds(0, _L)] * iv[0]
          for k in range(1, nd):
            acc = acc + rows_v[buf, row, pl.ds(k * _L, _L)] * iv[k]
          tr_v[j, :] = acc
        res = plsc.load_gather(tr_v, [lane, jnp.full((_L,), 0, jnp.int32)])
        for c in range(1, _L):
          res = res + plsc.load_gather(
              tr_v, [lane, jnp.full((_L,), c, jnp.int32)])
        sc_v[b, pl.ds(r * _L, _L)] = res

    def start(b, buf, sem):
      pltpu.async_copy(outw_hbm.at[idx_v.at[b]], rows_v.at[buf], sem)

    def wait(buf, sem):
      pltpu.make_async_copy(outw_hbm.at[idx_v.at[0]], rows_v.at[buf],
                            sem).wait()

    start(0, 0, sem0)
    start(1, 1, sem1)
    last = bpw - 1

    def body(i, _):
      g = 2 * i
      wait(0, sem0)
      compute(g, 0)
      start(jnp.minimum(g + 2, last), 0, sem0)
      wait(1, sem1)
      compute(g + 1, 1)
      start(jnp.minimum(g + 3, last), 1, sem1)
      return 0

    lax.fori_loop(0, half, body, 0)
    wait(0, sem0)
    wait(1, sem1)
    pltpu.sync_copy(sc_v, out_hbm.at[pl.ds(base, bpw), :])

  return sc_kernel(iword, idx_all, in_W, out_W)


def _log_sigmoid(x):
  return jnp.minimum(x, 0.0) - jnp.log1p(jnp.exp(-jnp.abs(x)))


def _tc_finish(scores, n_ctx, n_neg):
  B, K = scores.shape

  def body(s_ref, so_ref, sn_ref, loss_ref):
    s = s_ref[...]
    col = lax.broadcasted_iota(jnp.int32, s.shape, 1)
    ls_p = _log_sigmoid(s)
    ls_n = _log_sigmoid(-s)
    o = jnp.sum(jnp.where(col < n_ctx, ls_p, 0.0), axis=1) / n_ctx
    n = jnp.sum(jnp.where((col >= n_ctx) & (col < n_ctx + n_neg), ls_n, 0.0),
                axis=1) / n_ctx
    so_ref[...] = o
    sn_ref[...] = n
    loss_ref[0, 0] = -jnp.mean(o + n)

  return pl.pallas_call(
      body,
      out_shape=(
          jax.ShapeDtypeStruct((B,), jnp.float32),
          jax.ShapeDtypeStruct((B,), jnp.float32),
          jax.ShapeDtypeStruct((1, 1), jnp.float32),
      ),
  )(scores)


def kernel(iword, owords, nwords, in_W, out_W):
  B = iword.shape[0]
  pad = jnp.zeros((B, _K - _CTX - _NNEG), jnp.int32)
  idx_all = jnp.concatenate(
      [owords.astype(jnp.int32), nwords.astype(jnp.int32), pad], axis=1)
  scores = _sc_scores(iword.astype(jnp.int32), idx_all, in_W, out_W)
  score_o, score_n, loss = _tc_finish(scores, _CTX, _NNEG)
  return (loss[0, 0], score_o, score_n)


# speedup vs baseline: 1.4625x; 1.4625x over previous
import functools

import jax
import jax.numpy as jnp
from jax import lax
from jax.experimental import pallas as pl
from jax.experimental.pallas import tpu as pltpu
from jax.experimental.pallas import tpu_sc as plsc

_NC, _NS, _L = 2, 16, 16
_NW = _NC * _NS
_D = 128
_CTX = 20
_NNEG = 100
_K = 128


def _sc_scores(iword, idx_all, in_W, out_W):
  B = iword.shape[0]
  bpw = B // _NW
  half = bpw // 2
  nd = _D // _L
  ng = _K // _L

  mesh = plsc.VectorSubcoreMesh(core_axis_name="c", subcore_axis_name="s")

  @functools.partial(
      pl.kernel,
      out_type=jax.ShapeDtypeStruct((B, _K), jnp.float32),
      mesh=mesh,
      scratch_types=[
          pltpu.VMEM((bpw,), jnp.int32),
          pltpu.VMEM((bpw, _K), jnp.int32),
          pltpu.VMEM((bpw, _D), jnp.float32),
          pltpu.VMEM((2, _K, _D), jnp.float32),
          pltpu.VMEM((bpw, _K), jnp.float32),
          pltpu.SemaphoreType.DMA,
          pltpu.SemaphoreType.DMA,
          pltpu.SemaphoreType.DMA,
      ],
  )
  def sc_kernel(iword_hbm, idx_hbm, inw_hbm, outw_hbm, out_hbm,
                iw_v, idx_v, ivec_v, rows_v, sc_v, sem0, sem1, semi):
    wid = lax.axis_index("s") * _NC + lax.axis_index("c")
    base = wid * bpw
    pltpu.sync_copy(iword_hbm.at[pl.ds(base, bpw)], iw_v)
    pltpu.sync_copy(idx_hbm.at[pl.ds(base, bpw), :], idx_v)
    pltpu.async_copy(inw_hbm.at[iw_v], ivec_v, semi).wait()

    lane = lax.iota(jnp.int32, _L)
    dnums = lax.GatherDimensionNumbers(
        offset_dims=(), collapsed_slice_dims=(0,), start_index_map=(0,))
    rot_idx = [(lane + k) % _L for k in (8, 4, 2, 1)]

    def hsum(v):
      for idx in rot_idx:
        v = v + lax.gather(v, idx[:, None], dnums, slice_sizes=(1,),
                           mode=lax.GatherScatterMode.PROMISE_IN_BOUNDS)
      return v

    def compute(b, buf):
      iv = [ivec_v[b, pl.ds(k * _L, _L)] for k in range(nd)]
      for r in range(ng):
        res = jnp.zeros((_L,), jnp.float32)
        for j in range(_L):
          row = r * _L + j
          acc = rows_v[buf, row, pl.ds(0, _L)] * iv[0]
          for k in range(1, nd):
            acc = acc + rows_v[buf, row, pl.ds(k * _L, _L)] * iv[k]
          res = jnp.where(lane == j, hsum(acc), res)
        sc_v[b, pl.ds(r * _L, _L)] = res

    def start(b, buf, sem):
      pltpu.async_copy(outw_hbm.at[idx_v.at[b]], rows_v.at[buf], sem)

    def wait(buf, sem):
      pltpu.make_async_copy(outw_hbm.at[idx_v.at[0]], rows_v.at[buf],
                            sem).wait()

    start(0, 0, sem0)
    start(1, 1, sem1)
    last = bpw - 1

    def body(i, _):
      g = 2 * i
      wait(0, sem0)
      compute(g, 0)
      start(jnp.minimum(g + 2, last), 0, sem0)
      wait(1, sem1)
      compute(g + 1, 1)
      start(jnp.minimum(g + 3, last), 1, sem1)
      return 0

    lax.fori_loop(0, half, body, 0)
    wait(0, sem0)
    wait(1, sem1)
    pltpu.sync_copy(sc_v, out_hbm.at[pl.ds(base, bpw), :])

  return sc_kernel(iword, idx_all, in_W, out_W)


def _log_sigmoid(x):
  return jnp.minimum(x, 0.0) - jnp.log1p(jnp.exp(-jnp.abs(x)))


def _tc_finish(scores, n_ctx, n_neg):
  B, K = scores.shape

  def body(s_ref, so_ref, sn_ref, loss_ref):
    s = s_ref[...]
    col = lax.broadcasted_iota(jnp.int32, s.shape, 1)
    ls_p = _log_sigmoid(s)
    ls_n = _log_sigmoid(-s)
    o = jnp.sum(jnp.where(col < n_ctx, ls_p, 0.0), axis=1) / n_ctx
    n = jnp.sum(jnp.where((col >= n_ctx) & (col < n_ctx + n_neg), ls_n, 0.0),
                axis=1) / n_ctx
    so_ref[...] = o
    sn_ref[...] = n
    loss_ref[...] = jnp.full((1, 1), -1.0) * jnp.mean(o + n)

  return pl.pallas_call(
      body,
      out_shape=(
          jax.ShapeDtypeStruct((B,), jnp.float32),
          jax.ShapeDtypeStruct((B,), jnp.float32),
          jax.ShapeDtypeStruct((1, 1), jnp.float32),
      ),
  )(scores)


def kernel(iword, owords, nwords, in_W, out_W):
  B = iword.shape[0]
  pad = jnp.zeros((B, _K - _CTX - _NNEG), jnp.int32)
  idx_all = jnp.concatenate(
      [owords.astype(jnp.int32), nwords.astype(jnp.int32), pad], axis=1)
  scores = _sc_scores(iword.astype(jnp.int32), idx_all, in_W, out_W)
  score_o, score_n, loss = _tc_finish(scores, _CTX, _NNEG)
  return (loss[0, 0], score_o, score_n)
